# baseline (speedup 1.0000x reference)
import jax
import jax.numpy as jnp
from jax.experimental import pallas as pl
from jax.experimental.pallas import tpu as pltpu

_TM = 256
_NBUF = 4


def _body(x_hbm, w_ref, b_ref, o_ref, xbuf, sems):
    m, k = x_hbm.shape
    nt = m // _TM

    def copy(tile, slot):
        return pltpu.make_async_copy(
            x_hbm.at[pl.ds(tile * _TM, _TM), :], xbuf.at[slot], sems.at[slot]
        )

    for s in range(min(_NBUF, nt)):
        copy(s, s).start()

    for i in range(nt):
        slot = i % _NBUF
        copy(i, slot).wait()
        acc = jax.lax.dot_general(
            xbuf[slot],
            w_ref[...],
            dimension_numbers=(((1,), (1,)), ((), ())),
            preferred_element_type=jnp.float32,
        )
        o_ref[pl.ds(i * _TM, _TM), :] = acc + b_ref[...]
        if i + _NBUF < nt:
            copy(i + _NBUF, slot).start()


@jax.jit
def kernel(input, weight, bias):
    m, k = input.shape
    n = weight.shape[0]
    return pl.pallas_call(
        _body,
        in_specs=[
            pl.BlockSpec(memory_space=pl.ANY),
            pl.BlockSpec((n, k), lambda: (0, 0)),
            pl.BlockSpec((1, n), lambda: (0, 0)),
        ],
        out_specs=pl.BlockSpec((m, n), lambda: (0, 0)),
        out_shape=jax.ShapeDtypeStruct((m, n), jnp.float32),
        scratch_shapes=[
            pltpu.VMEM((_NBUF, _TM, k), jnp.float32),
            pltpu.SemaphoreType.DMA((_NBUF,)),
        ],
    )(input, weight, bias.reshape(1, n))
